# baseline (speedup 1.0000x reference)
import functools

import jax
import jax.numpy as jnp
from jax import lax
from jax.experimental import pallas as pl
from jax.experimental.pallas import tpu as pltpu
from jax.experimental.pallas import tpu_sc as plsc

N = 10000
E = 320000
D = 128
DE = 16
STEPS = 3

NC = 2
NS = 16
NW = NC * NS
EW = E // NW
K = 125
CH = EW // K
NBUF = 2
NP = 10240
NPT = NP // NS
E8 = E * DE // 128

BT = 2000
BN = 2000

_F32 = jnp.float32


def _edge_pre_body(ea_ref, bde_ref, bda_ref, be8_ref, ce_ref, t0_ref):
    ea = ea_ref[...].reshape(BT, 128)
    ce = jnp.dot(ea, bde_ref[...], preferred_element_type=_F32) + be8_ref[...]
    ce_ref[...] = ce.reshape(BT * 128)
    t0 = ce + jnp.dot(ea, bda_ref[...], preferred_element_type=_F32)
    t0_ref[...] = t0.reshape(BT * 128)


def _edge_pre(eaf, bd_e, bd_a, be8):
    nb = E8 // BT
    return pl.pallas_call(
        _edge_pre_body,
        grid=(nb,),
        in_specs=[
            pl.BlockSpec((BT * 128,), lambda i: (i,)),
            pl.BlockSpec((128, 128), lambda i: (0, 0)),
            pl.BlockSpec((128, 128), lambda i: (0, 0)),
            pl.BlockSpec((1, 128), lambda i: (0, 0)),
        ],
        out_specs=(
            pl.BlockSpec((BT * 128,), lambda i: (i,)),
            pl.BlockSpec((BT * 128,), lambda i: (i,)),
        ),
        out_shape=(
            jax.ShapeDtypeStruct((E * DE,), _F32),
            jax.ShapeDtypeStruct((E * DE,), _F32),
        ),
    )(eaf, bd_e, bd_a, be8)


def _edge_t_body(msg_ref, ce_ref, bda_ref, t_ref):
    m = msg_ref[...].reshape(BT, 128)
    t = (jnp.dot(m, bda_ref[...], preferred_element_type=_F32)
         + ce_ref[...].reshape(BT, 128))
    t_ref[...] = t.reshape(BT * 128)


def _edge_t(msgf, cef, bd_a):
    nb = E8 // BT
    return pl.pallas_call(
        _edge_t_body,
        grid=(nb,),
        in_specs=[
            pl.BlockSpec((BT * 128,), lambda i: (i,)),
            pl.BlockSpec((BT * 128,), lambda i: (i,)),
            pl.BlockSpec((128, 128), lambda i: (0, 0)),
        ],
        out_specs=pl.BlockSpec((BT * 128,), lambda i: (i,)),
        out_shape=jax.ShapeDtypeStruct((E * DE,), _F32),
    )(msgf, cef, bd_a)


def _node_pre_body(x_ref, wne_ref, bn_ref, wer_ref, wec_ref,
                   cn_ref, pr_ref, pc_ref):
    x = x_ref[...]
    cn_ref[...] = jnp.dot(x, wne_ref[...], preferred_element_type=_F32) + bn_ref[...]
    pr_ref[...] = jnp.dot(x, wer_ref[...], preferred_element_type=_F32)
    pc_ref[...] = jnp.dot(x, wec_ref[...], preferred_element_type=_F32)


def _node_pre(x, wn_e, bn, we_r, we_c):
    nb = N // BN
    return pl.pallas_call(
        _node_pre_body,
        grid=(nb,),
        in_specs=[
            pl.BlockSpec((BN, D), lambda i: (i, 0)),
            pl.BlockSpec((D, D), lambda i: (0, 0)),
            pl.BlockSpec((1, D), lambda i: (0, 0)),
            pl.BlockSpec((D, DE), lambda i: (0, 0)),
            pl.BlockSpec((D, DE), lambda i: (0, 0)),
        ],
        out_specs=(
            pl.BlockSpec((BN, D), lambda i: (i, 0)),
            pl.BlockSpec((BN, DE), lambda i: (i, 0)),
            pl.BlockSpec((BN, DE), lambda i: (i, 0)),
        ),
        out_shape=(
            jax.ShapeDtypeStruct((N, D), _F32),
            jax.ShapeDtypeStruct((N, DE), _F32),
            jax.ShapeDtypeStruct((N, DE), _F32),
        ),
    )(x, wn_e, bn, we_r, we_c)


def _node_body(x_ref, a2_ref, cn_ref, wnx_ref, wna_ref, wer_ref, wec_ref,
               xo_ref, pr_ref, pc_ref):
    x = x_ref[...]
    agg = a2_ref[0] + a2_ref[1]
    h = jnp.dot(x, wnx_ref[...], preferred_element_type=_F32)
    h = h + jnp.dot(agg, wna_ref[...], preferred_element_type=_F32)
    h = h + cn_ref[...]
    xo = jnp.maximum(h, 0.0)
    xo_ref[...] = xo
    pr_ref[...] = jnp.dot(xo, wer_ref[...], preferred_element_type=_F32)
    pc_ref[...] = jnp.dot(xo, wec_ref[...], preferred_element_type=_F32)


def _node(x, agg2, cn, wn_x, wn_a, we_r, we_c):
    nb = N // BN
    return pl.pallas_call(
        _node_body,
        grid=(nb,),
        in_specs=[
            pl.BlockSpec((BN, D), lambda i: (i, 0)),
            pl.BlockSpec((NC, BN, DE), lambda i: (0, i, 0)),
            pl.BlockSpec((BN, D), lambda i: (i, 0)),
            pl.BlockSpec((D, D), lambda i: (0, 0)),
            pl.BlockSpec((DE, D), lambda i: (0, 0)),
            pl.BlockSpec((D, DE), lambda i: (0, 0)),
            pl.BlockSpec((D, DE), lambda i: (0, 0)),
        ],
        out_specs=(
            pl.BlockSpec((BN, D), lambda i: (i, 0)),
            pl.BlockSpec((BN, DE), lambda i: (i, 0)),
            pl.BlockSpec((BN, DE), lambda i: (i, 0)),
        ),
        out_shape=(
            jax.ShapeDtypeStruct((N, D), _F32),
            jax.ShapeDtypeStruct((N, DE), _F32),
            jax.ShapeDtypeStruct((N, DE), _F32),
        ),
    )(x, agg2, cn, wn_x, wn_a, we_r, we_c)


def _sc_edge_body(row_ref, col_ref, pr_ref, pc_ref, t_ref,
                  msg_ref, agg_ref,
                  rowv, colv, av, bv, tv, mv, mvf, zb, agg_sh,
                  sem0, sem1, osem0, osem1):
    c = lax.axis_index("c")
    s = lax.axis_index("s")
    wid = s * NC + c
    ebase = wid * EW

    pltpu.sync_copy(row_ref.at[wid], rowv)
    pltpu.sync_copy(col_ref.at[wid], colv)

    def _zset(j, carry):
        zb[j] = jnp.zeros((DE,), _F32)
        return carry
    lax.fori_loop(0, NPT, _zset, None)
    nodebase = s * NPT
    pltpu.sync_copy(zb, agg_sh.at[pl.ds(nodebase, NPT)])
    plsc.subcore_barrier()

    sems = (sem0, sem1)
    osems = (osem0, osem1)

    def in_copies(ch, b):
        f0 = (ebase + ch * K) * DE
        return (
            pltpu.make_async_copy(t_ref.at[pl.ds(f0, K * DE)], tv.at[b],
                                  sems[b]),
            pltpu.make_async_copy(pr_ref.at[rowv.at[ch]], av.at[b], sems[b]),
            pltpu.make_async_copy(pc_ref.at[colv.at[ch]], bv.at[b], sems[b]),
        )

    def out_copies(ch, b):
        f0 = (ebase + ch * K) * DE
        return (
            pltpu.make_async_copy(mvf.at[b], msg_ref.at[pl.ds(f0, K * DE)],
                                  osems[b]),
        )

    def start_outs(ch, b):
        f0 = (ebase + ch * K) * DE
        pltpu.async_copy(mvf.at[b], msg_ref.at[pl.ds(f0, K * DE)], osems[b])
        pltpu.sync_copy(mv.at[b], agg_sh.at[colv.at[ch]], add=True)

    def compute(b):
        def _crow(j, cc):
            v = av[b, j] + bv[b, j] + tv[b, pl.ds(j * DE, DE)]
            m = jnp.maximum(v, 0.0)
            mv[b, j] = m
            mvf[b, pl.ds(j * DE, DE)] = m
            return cc
        lax.fori_loop(0, K, _crow, None, unroll=5)

    for d in in_copies(0, 0):
        d.start()
    for d in in_copies(1, 1):
        d.start()

    for b in range(2):
        for d in in_copies(b, b):
            d.wait()
        compute(b)
        for d in in_copies(b + 2, b):
            d.start()
        start_outs(b, b)

    def outer(i, carry):
        for b in range(2):
            ch = 2 * i + b
            for d in in_copies(ch, b):
                d.wait()
            for d in out_copies(ch - 2, b):
                d.wait()
            compute(b)
            for d in in_copies(ch + 2, b):
                d.start()
            start_outs(ch, b)
        return carry

    lax.fori_loop(1, CH // 2 - 1, outer, None)

    for b in range(2):
        ch = CH - 2 + b
        for d in in_copies(ch, b):
            d.wait()
        for d in out_copies(ch - 2, b):
            d.wait()
        compute(b)
        start_outs(ch, b)
    for b in range(2):
        for d in out_copies(CH - 2 + b, b):
            d.wait()
    plsc.subcore_barrier()

    pltpu.sync_copy(agg_sh.at[pl.ds(nodebase, NPT)], zb)
    pltpu.sync_copy(zb, agg_ref.at[c, pl.ds(nodebase, NPT)])


@functools.cache
def _make_sc_edge():
    return pl.kernel(
        _sc_edge_body,
        out_type=(
            jax.ShapeDtypeStruct((E * DE,), _F32),
            jax.ShapeDtypeStruct((NC, NP, DE), _F32),
        ),
        mesh=plsc.VectorSubcoreMesh(core_axis_name="c", subcore_axis_name="s",
                                    num_cores=NC, num_subcores=NS),
        compiler_params=pltpu.CompilerParams(use_tc_tiling_on_sc=False),
        scratch_types=[
            pltpu.VMEM((CH, K), jnp.int32),
            pltpu.VMEM((CH, K), jnp.int32),
            pltpu.VMEM((NBUF, K, DE), _F32),
            pltpu.VMEM((NBUF, K, DE), _F32),
            pltpu.VMEM((NBUF, K * DE), _F32),
            pltpu.VMEM((NBUF, K, DE), _F32),
            pltpu.VMEM((NBUF, K * DE), _F32),
            pltpu.VMEM((NPT, DE), _F32),
            pltpu.VMEM_SHARED((NP, DE), _F32),
            pltpu.SemaphoreType.DMA,
            pltpu.SemaphoreType.DMA,
            pltpu.SemaphoreType.DMA,
            pltpu.SemaphoreType.DMA,
        ],
    )


def _sc_edge(row3, col3, prt, pct, tt):
    return _make_sc_edge()(row3, col3, prt, pct, tt)


def kernel(x, edge_index, edge_attr, W_e, b_e, W_n, b_n):
    row = edge_index[0].reshape(NW, CH, K)
    col = edge_index[1].reshape(NW, CH, K)

    we_r = W_e[0:D]
    we_c = W_e[D:2 * D]
    we_a = W_e[2 * D:2 * D + DE]
    we_e = W_e[2 * D + DE:]
    eye8 = jnp.eye(8, dtype=_F32)
    bd_a = jnp.kron(eye8, we_a)
    bd_e = jnp.kron(eye8, we_e)
    be8 = jnp.tile(b_e, 8).reshape(1, 128)

    wn_x = W_n[0:D]
    wn_a = W_n[D:D + DE]
    wn_e = W_n[D + DE:]

    eaf = edge_attr.reshape(E * DE)
    cef, t = _edge_pre(eaf, bd_e, bd_a, be8)
    cn, pr, pc = _node_pre(x, wn_e, b_n.reshape(1, D), we_r, we_c)

    msg = None
    for step in range(STEPS):
        msg, agg2 = _sc_edge(row, col, pr, pc, t)
        x, pr, pc = _node(x, agg2, cn, wn_x, wn_a, we_r, we_c)
        if step < STEPS - 1:
            t = _edge_t(msg, cef, bd_a)
    return (x, msg.reshape(E, DE))


# speedup vs baseline: 1.1136x; 1.1136x over previous
import functools

import jax
import jax.numpy as jnp
from jax import lax
from jax.experimental import pallas as pl
from jax.experimental.pallas import tpu as pltpu
from jax.experimental.pallas import tpu_sc as plsc

N = 10000
E = 320000
D = 128
DE = 16
STEPS = 3

NC = 2
NS = 16
NW = NC * NS
EW = E // NW
K = 125
CH = EW // K
NBUF = 2
NP = 10240
NPT = NP // NS
E8 = E * DE // 128

BT = 2000
BN = 2000

_F32 = jnp.float32


def _edge_pre_body(ea_ref, bde_ref, bda_ref, be8_ref, ce_ref, t0_ref):
    ea = ea_ref[...].reshape(BT, 128)
    ce = jnp.dot(ea, bde_ref[...], preferred_element_type=_F32) + be8_ref[...]
    ce_ref[...] = ce.reshape(BT * 128)
    t0 = ce + jnp.dot(ea, bda_ref[...], preferred_element_type=_F32)
    t0_ref[...] = t0.reshape(BT * 128)


def _edge_pre(eaf, bd_e, bd_a, be8):
    nb = E8 // BT
    return pl.pallas_call(
        _edge_pre_body,
        grid=(nb,),
        in_specs=[
            pl.BlockSpec((BT * 128,), lambda i: (i,)),
            pl.BlockSpec((128, 128), lambda i: (0, 0)),
            pl.BlockSpec((128, 128), lambda i: (0, 0)),
            pl.BlockSpec((1, 128), lambda i: (0, 0)),
        ],
        out_specs=(
            pl.BlockSpec((BT * 128,), lambda i: (i,)),
            pl.BlockSpec((BT * 128,), lambda i: (i,)),
        ),
        out_shape=(
            jax.ShapeDtypeStruct((E * DE,), _F32),
            jax.ShapeDtypeStruct((E * DE,), _F32),
        ),
    )(eaf, bd_e, bd_a, be8)


def _edge_t_body(msg_ref, ce_ref, bda_ref, t_ref):
    m = msg_ref[...].reshape(BT, 128)
    t = (jnp.dot(m, bda_ref[...], preferred_element_type=_F32)
         + ce_ref[...].reshape(BT, 128))
    t_ref[...] = t.reshape(BT * 128)


def _edge_t(msgf, cef, bd_a):
    nb = E8 // BT
    return pl.pallas_call(
        _edge_t_body,
        grid=(nb,),
        in_specs=[
            pl.BlockSpec((BT * 128,), lambda i: (i,)),
            pl.BlockSpec((BT * 128,), lambda i: (i,)),
            pl.BlockSpec((128, 128), lambda i: (0, 0)),
        ],
        out_specs=pl.BlockSpec((BT * 128,), lambda i: (i,)),
        out_shape=jax.ShapeDtypeStruct((E * DE,), _F32),
    )(msgf, cef, bd_a)


def _node_pre_body(x_ref, wne_ref, bn_ref, wer_ref, wec_ref,
                   cn_ref, pr_ref, pc_ref):
    x = x_ref[...]
    cn_ref[...] = jnp.dot(x, wne_ref[...], preferred_element_type=_F32) + bn_ref[...]
    pr_ref[...] = jnp.dot(x, wer_ref[...], preferred_element_type=_F32)
    pc_ref[...] = jnp.dot(x, wec_ref[...], preferred_element_type=_F32)


def _node_pre(x, wn_e, bn, we_r, we_c):
    nb = N // BN
    return pl.pallas_call(
        _node_pre_body,
        grid=(nb,),
        in_specs=[
            pl.BlockSpec((BN, D), lambda i: (i, 0)),
            pl.BlockSpec((D, D), lambda i: (0, 0)),
            pl.BlockSpec((1, D), lambda i: (0, 0)),
            pl.BlockSpec((D, DE), lambda i: (0, 0)),
            pl.BlockSpec((D, DE), lambda i: (0, 0)),
        ],
        out_specs=(
            pl.BlockSpec((BN, D), lambda i: (i, 0)),
            pl.BlockSpec((BN, DE), lambda i: (i, 0)),
            pl.BlockSpec((BN, DE), lambda i: (i, 0)),
        ),
        out_shape=(
            jax.ShapeDtypeStruct((N, D), _F32),
            jax.ShapeDtypeStruct((N, DE), _F32),
            jax.ShapeDtypeStruct((N, DE), _F32),
        ),
    )(x, wn_e, bn, we_r, we_c)


def _node_body(x_ref, a2_ref, cn_ref, wnx_ref, wna_ref, wer_ref, wec_ref,
               xo_ref, pr_ref, pc_ref):
    x = x_ref[...]
    agg = a2_ref[0] + a2_ref[1]
    h = jnp.dot(x, wnx_ref[...], preferred_element_type=_F32)
    h = h + jnp.dot(agg, wna_ref[...], preferred_element_type=_F32)
    h = h + cn_ref[...]
    xo = jnp.maximum(h, 0.0)
    xo_ref[...] = xo
    pr_ref[...] = jnp.dot(xo, wer_ref[...], preferred_element_type=_F32)
    pc_ref[...] = jnp.dot(xo, wec_ref[...], preferred_element_type=_F32)


def _node(x, agg2, cn, wn_x, wn_a, we_r, we_c):
    nb = N // BN
    return pl.pallas_call(
        _node_body,
        grid=(nb,),
        in_specs=[
            pl.BlockSpec((BN, D), lambda i: (i, 0)),
            pl.BlockSpec((NC, BN, DE), lambda i: (0, i, 0)),
            pl.BlockSpec((BN, D), lambda i: (i, 0)),
            pl.BlockSpec((D, D), lambda i: (0, 0)),
            pl.BlockSpec((DE, D), lambda i: (0, 0)),
            pl.BlockSpec((D, DE), lambda i: (0, 0)),
            pl.BlockSpec((D, DE), lambda i: (0, 0)),
        ],
        out_specs=(
            pl.BlockSpec((BN, D), lambda i: (i, 0)),
            pl.BlockSpec((BN, DE), lambda i: (i, 0)),
            pl.BlockSpec((BN, DE), lambda i: (i, 0)),
        ),
        out_shape=(
            jax.ShapeDtypeStruct((N, D), _F32),
            jax.ShapeDtypeStruct((N, DE), _F32),
            jax.ShapeDtypeStruct((N, DE), _F32),
        ),
    )(x, agg2, cn, wn_x, wn_a, we_r, we_c)


def _sc_edge_body(row_ref, col_ref, pr_ref, pc_ref, t_ref,
                  msg_ref, agg_ref,
                  rowv, colv, av, bv, tv, mv, mvf, zb, agg_sh,
                  sem0, sem1, osem0, osem1):
    c = lax.axis_index("c")
    s = lax.axis_index("s")
    wid = s * NC + c
    ebase = wid * EW

    pltpu.sync_copy(row_ref.at[wid], rowv)
    pltpu.sync_copy(col_ref.at[wid], colv)

    def _zset(j, carry):
        zb[j] = jnp.zeros((DE,), _F32)
        return carry
    lax.fori_loop(0, NPT, _zset, None)
    nodebase = s * NPT
    pltpu.sync_copy(zb, agg_sh.at[pl.ds(nodebase, NPT)])
    plsc.subcore_barrier()

    sems = (sem0, sem1)
    osems = (osem0, osem1)

    def in_copies(ch, b):
        f0 = (ebase + ch * K) * DE
        return (
            pltpu.make_async_copy(t_ref.at[pl.ds(f0, K * DE)], tv.at[b],
                                  sems[b]),
            pltpu.make_async_copy(pr_ref.at[rowv.at[ch]], av.at[b], sems[b]),
            pltpu.make_async_copy(pc_ref.at[colv.at[ch]], bv.at[b], sems[b]),
        )

    def out_copies(ch, b):
        f0 = (ebase + ch * K) * DE
        return (
            pltpu.make_async_copy(mvf.at[b], msg_ref.at[pl.ds(f0, K * DE)],
                                  osems[b]),
        )

    def start_outs(ch, b):
        f0 = (ebase + ch * K) * DE
        pltpu.async_copy(mvf.at[b], msg_ref.at[pl.ds(f0, K * DE)], osems[b])
        pltpu.sync_copy(mv.at[b], agg_sh.at[colv.at[ch]], add=True)

    def compute(b):
        def _crow(j, cc):
            v = av[b, j] + bv[b, j] + tv[b, pl.ds(j * DE, DE)]
            m = jnp.maximum(v, 0.0)
            mv[b, j] = m
            mvf[b, pl.ds(j * DE, DE)] = m
            return cc
        lax.fori_loop(0, K, _crow, None)

    for d in in_copies(0, 0):
        d.start()
    for d in in_copies(1, 1):
        d.start()

    for b in range(2):
        for d in in_copies(b, b):
            d.wait()
        compute(b)
        for d in in_copies(b + 2, b):
            d.start()
        start_outs(b, b)

    def outer(i, carry):
        for b in range(2):
            ch = 2 * i + b
            for d in in_copies(ch, b):
                d.wait()
            for d in out_copies(ch - 2, b):
                d.wait()
            compute(b)
            for d in in_copies(ch + 2, b):
                d.start()
            start_outs(ch, b)
        return carry

    lax.fori_loop(1, CH // 2 - 1, outer, None)

    for b in range(2):
        ch = CH - 2 + b
        for d in in_copies(ch, b):
            d.wait()
        for d in out_copies(ch - 2, b):
            d.wait()
        compute(b)
        start_outs(ch, b)
    for b in range(2):
        for d in out_copies(CH - 2 + b, b):
            d.wait()
    plsc.subcore_barrier()

    pltpu.sync_copy(agg_sh.at[pl.ds(nodebase, NPT)], zb)
    pltpu.sync_copy(zb, agg_ref.at[c, pl.ds(nodebase, NPT)])


@functools.cache
def _make_sc_edge():
    return pl.kernel(
        _sc_edge_body,
        out_type=(
            jax.ShapeDtypeStruct((E * DE,), _F32),
            jax.ShapeDtypeStruct((NC, NP, DE), _F32),
        ),
        mesh=plsc.VectorSubcoreMesh(core_axis_name="c", subcore_axis_name="s",
                                    num_cores=NC, num_subcores=NS),
        compiler_params=pltpu.CompilerParams(use_tc_tiling_on_sc=False),
        scratch_types=[
            pltpu.VMEM((CH, K), jnp.int32),
            pltpu.VMEM((CH, K), jnp.int32),
            pltpu.VMEM((NBUF, K, DE), _F32),
            pltpu.VMEM((NBUF, K, DE), _F32),
            pltpu.VMEM((NBUF, K * DE), _F32),
            pltpu.VMEM((NBUF, K, DE), _F32),
            pltpu.VMEM((NBUF, K * DE), _F32),
            pltpu.VMEM((NPT, DE), _F32),
            pltpu.VMEM_SHARED((NP, DE), _F32),
            pltpu.SemaphoreType.DMA,
            pltpu.SemaphoreType.DMA,
            pltpu.SemaphoreType.DMA,
            pltpu.SemaphoreType.DMA,
        ],
    )


def _sc_edge(row3, col3, prt, pct, tt):
    return _make_sc_edge()(row3, col3, prt, pct, tt)


def kernel(x, edge_index, edge_attr, W_e, b_e, W_n, b_n):
    row = edge_index[0].reshape(NW, CH, K)
    col = edge_index[1].reshape(NW, CH, K)

    we_r = W_e[0:D]
    we_c = W_e[D:2 * D]
    we_a = W_e[2 * D:2 * D + DE]
    we_e = W_e[2 * D + DE:]
    eye8 = jnp.eye(8, dtype=_F32)
    bd_a = jnp.kron(eye8, we_a)
    bd_e = jnp.kron(eye8, we_e)
    be8 = jnp.tile(b_e, 8).reshape(1, 128)

    wn_x = W_n[0:D]
    wn_a = W_n[D:D + DE]
    wn_e = W_n[D + DE:]

    eaf = edge_attr.reshape(E * DE)
    cef, t = _edge_pre(eaf, bd_e, bd_a, be8)
    cn, pr, pc = _node_pre(x, wn_e, b_n.reshape(1, D), we_r, we_c)

    msg = None
    for step in range(STEPS):
        msg, agg2 = _sc_edge(row, col, pr, pc, t)
        x, pr, pc = _node(x, agg2, cn, wn_x, wn_a, we_r, we_c)
        if step < STEPS - 1:
            t = _edge_t(msg, cef, bd_a)
    return (x, msg.reshape(E, DE))
